# Initial kernel scaffold; baseline (speedup 1.0000x reference)
import jax
import jax.numpy as jnp
from jax.experimental import pallas as pl


def kernel(x, edge_index, W_in, b_in, prototypes, W_l, b_l, W_r, b_r, att, gat_bias, W_cls, b_cls):
    raise NotImplementedError("write your pallas kernel here")


# speedup vs baseline: 3.6189x; 3.6189x over previous
import functools

import jax
import jax.numpy as jnp
from jax import lax
from jax.experimental import pallas as pl
from jax.experimental.pallas import tpu as pltpu
from jax.experimental.pallas import tpu_sc as plsc

N_NODES = 10000
D_IN = 128
HID = 128
HEADS = 4
DM = HEADS * HID
NEG = 0.2

NP = 10240
ROWS_BLK = 1280
N_BLKS = NP // ROWS_BLK

NC, NS, L = 2, 16, 16
NW = NC * NS
EP_TILE = 5376
E_PAD = NW * EP_TILE
B = 64
NB = EP_TILE // B
G = B // L
ZR = 256
ZR2 = 32


def _dense_pre_body(x_ref, w_in_ref, b_in_ref, pro_ref, wlm_ref, wls_ref,
                    bl_ref, wrm_ref, wrs_ref, br_ref, att_ref,
                    xl_ref, xr_ref, aself_ref):
    xb = x_ref[...]
    h = jnp.maximum(xb @ w_in_ref[...] + b_in_ref[...], 0.0)
    hn = h / (jnp.sqrt(jnp.sum(h * h, axis=1, keepdims=True)) + 1e-12)
    pro = pro_ref[...]
    pn = pro / (jnp.sqrt(jnp.sum(pro * pro, axis=1, keepdims=True)) + 1e-12)
    sem = hn @ pn.T
    xl = h @ wlm_ref[...] + sem @ wls_ref[...] + bl_ref[...]
    xr = h @ wrm_ref[...] + sem @ wrs_ref[...] + br_ref[...]
    xl_ref[...] = xl
    xr_ref[...] = xr
    s = xl + xr
    lr = jnp.maximum(s, NEG * s)
    att = att_ref[...]
    parts = [
        jnp.sum(lr[:, h_ * HID:(h_ + 1) * HID] * att[h_][None, :],
                axis=1, keepdims=True)
        for h_ in range(HEADS)
    ]
    parts.append(jnp.zeros((ROWS_BLK, 16 - HEADS), jnp.float32))
    aself_ref[...] = jnp.concatenate(parts, axis=1)


def _dense_pre(xp, w_in, b_in, pro, wlm, wls, bl, wrm, wrs, br, att):
    full = lambda i: (0, 0)
    blk = lambda i: (i, 0)
    return pl.pallas_call(
        _dense_pre_body,
        grid=(N_BLKS,),
        in_specs=[
            pl.BlockSpec((ROWS_BLK, D_IN), blk),
            pl.BlockSpec((D_IN, HID), full),
            pl.BlockSpec((1, HID), full),
            pl.BlockSpec((2, HID), full),
            pl.BlockSpec((HID, DM), full),
            pl.BlockSpec((2, DM), full),
            pl.BlockSpec((1, DM), full),
            pl.BlockSpec((HID, DM), full),
            pl.BlockSpec((2, DM), full),
            pl.BlockSpec((1, DM), full),
            pl.BlockSpec((HEADS, HID), full),
        ],
        out_specs=[
            pl.BlockSpec((ROWS_BLK, DM), blk),
            pl.BlockSpec((ROWS_BLK, DM), blk),
            pl.BlockSpec((ROWS_BLK, 16), blk),
        ],
        out_shape=[
            jax.ShapeDtypeStruct((NP, DM), jnp.float32),
            jax.ShapeDtypeStruct((NP, DM), jnp.float32),
            jax.ShapeDtypeStruct((NP, 16), jnp.float32),
        ],
    )(xp, w_in, b_in, pro, wlm, wls, bl, wrm, wrs, br, att)


def _sc_pass1_body(src_hbm, dst_hbm, xl_hbm, xr_hbm, aself_hbm, attb_hbm,
                   aexp_hbm, den_hbm,
                   src_v, dst_v, xl_rows, xr_rows, aself_rows, aexp_buf,
                   attb_v, zbuf, den_sh, sem1, sem2, sem3):
    cid = lax.axis_index("c")
    sid = lax.axis_index("s")
    wid = cid * NS + sid

    pltpu.sync_copy(attb_hbm, attb_v)

    zero16 = jnp.zeros((L,), jnp.float32)

    @pl.loop(0, B)
    def _(i):
        aexp_buf[i, :] = zero16

    @pl.loop(0, ZR)
    def _(i):
        zbuf[i, :] = zero16

    @pl.when(sid == 0)
    def _():
        @pl.loop(0, NP // ZR)
        def _(k):
            pltpu.sync_copy(zbuf, den_sh.at[pl.ds(k * ZR, ZR)])

    plsc.subcore_barrier()

    riota = lax.iota(jnp.int32, L)
    rows = [riota + g * L for g in range(G)]

    @pl.loop(0, NB)
    def _(b):
        base = wid * EP_TILE + b * B
        pltpu.sync_copy(src_hbm.at[pl.ds(base, B)], src_v)
        pltpu.sync_copy(dst_hbm.at[pl.ds(base, B)], dst_v)
        cp1 = pltpu.async_copy(xl_hbm.at[src_v], xl_rows, sem1)
        cp2 = pltpu.async_copy(xr_hbm.at[dst_v], xr_rows, sem2)
        cp3 = pltpu.async_copy(aself_hbm.at[dst_v], aself_rows, sem3)
        cp1.wait()
        cp2.wait()
        cp3.wait()

        for h in range(HEADS):
            init = tuple(jnp.zeros((L,), jnp.float32) for _ in range(G))

            @pl.loop(0, HID, init_carry=init, unroll=2)
            def accs(c, accs):
                attv = attb_v[pl.ds((h * HID + c) * L, L)]
                col = jnp.full((L,), h * HID, jnp.int32) + c
                out = []
                for g in range(G):
                    xlv = plsc.load_gather(xl_rows, [rows[g], col])
                    xrv = plsc.load_gather(xr_rows, [rows[g], col])
                    s = xlv + xrv
                    lr = jnp.maximum(s, NEG * s)
                    out.append(accs[g] + attv * lr)
                return tuple(out)

            hcol = jnp.full((L,), h, jnp.int32)
            for g in range(G):
                aself_v = plsc.load_gather(aself_rows, [rows[g], hcol])
                av = jnp.exp(accs[g] - aself_v)
                plsc.store_scatter(aexp_buf, [rows[g], hcol], av)

        pltpu.sync_copy(aexp_buf, aexp_hbm.at[pl.ds(base, B)])
        pltpu.sync_copy(aexp_buf, den_sh.at[dst_v], add=True)

    plsc.subcore_barrier()

    @pl.when(sid == 0)
    def _():
        pltpu.sync_copy(den_sh, den_hbm.at[cid])


def _sc_pass1(src, dst, xl, xr, aself, attb):
    mesh = plsc.VectorSubcoreMesh(core_axis_name="c", subcore_axis_name="s",
                                  num_cores=NC, num_subcores=NS)
    return pl.kernel(
        _sc_pass1_body,
        out_type=[
            jax.ShapeDtypeStruct((E_PAD, 16), jnp.float32),
            jax.ShapeDtypeStruct((NC, NP, 16), jnp.float32),
        ],
        mesh=mesh,
        compiler_params=pltpu.CompilerParams(use_tc_tiling_on_sc=False, needs_layout_passes=False),
        scratch_types=[
            pltpu.VMEM((B,), jnp.int32),
            pltpu.VMEM((B,), jnp.int32),
            pltpu.VMEM((B, DM), jnp.float32),
            pltpu.VMEM((B, DM), jnp.float32),
            pltpu.VMEM((B, 16), jnp.float32),
            pltpu.VMEM((B, 16), jnp.float32),
            pltpu.VMEM((HEADS * HID * L,), jnp.float32),
            pltpu.VMEM((ZR, 16), jnp.float32),
            pltpu.VMEM_SHARED((NP, 16), jnp.float32),
            pltpu.SemaphoreType.DMA,
            pltpu.SemaphoreType.DMA,
            pltpu.SemaphoreType.DMA,
        ],
    )(src, dst, xl, xr, aself, attb)


def _sc_pass2_body(src_hbm, dst_hbm, xl_hbm, aexp_hbm, den_hbm,
                   outp_hbm,
                   src_v, dst_v, xl_rows, aexp_rows, den0, den1, m_rows,
                   zbuf, out_sh, sem1, sem2, sem3):
    cid = lax.axis_index("c")
    sid = lax.axis_index("s")
    wid = cid * NS + sid

    zero128 = jnp.zeros((L,), jnp.float32)

    @pl.loop(0, ZR2)
    def _(i):
        for k in range(HID // L):
            zbuf[i, pl.ds(k * L, L)] = zero128

    @pl.when(sid == 0)
    def _():
        @pl.loop(0, NP // ZR2)
        def _(k):
            pltpu.sync_copy(zbuf, out_sh.at[pl.ds(k * ZR2, ZR2)])

    plsc.subcore_barrier()

    riota = lax.iota(jnp.int32, L)
    rows = [riota + g * L for g in range(G)]

    @pl.loop(0, NB)
    def _(b):
        base = wid * EP_TILE + b * B
        pltpu.sync_copy(src_hbm.at[pl.ds(base, B)], src_v)
        pltpu.sync_copy(dst_hbm.at[pl.ds(base, B)], dst_v)
        pltpu.sync_copy(aexp_hbm.at[pl.ds(base, B)], aexp_rows)
        cp1 = pltpu.async_copy(xl_hbm.at[src_v], xl_rows, sem1)
        cp2 = pltpu.async_copy(den_hbm.at[0].at[dst_v], den0, sem2)
        cp3 = pltpu.async_copy(den_hbm.at[1].at[dst_v], den1, sem3)
        cp1.wait()
        cp2.wait()
        cp3.wait()

        for g in range(G):
            wv = []
            for h in range(HEADS):
                hcol = jnp.full((L,), h, jnp.int32)
                av = plsc.load_gather(aexp_rows, [rows[g], hcol])
                d0 = plsc.load_gather(den0, [rows[g], hcol])
                d1 = plsc.load_gather(den1, [rows[g], hcol])
                wv.append(av / (d0 + d1 + 1e-16))

            @pl.loop(0, HID, unroll=2)
            def _(c):
                acc = jnp.zeros((L,), jnp.float32)
                for h in range(HEADS):
                    col = jnp.full((L,), h * HID, jnp.int32) + c
                    xlv = plsc.load_gather(xl_rows, [rows[g], col])
                    acc = acc + wv[h] * xlv
                plsc.store_scatter(m_rows, [rows[g],
                                            jnp.zeros((L,), jnp.int32) + c],
                                   acc)

        pltpu.sync_copy(m_rows, out_sh.at[dst_v], add=True)

    plsc.subcore_barrier()

    @pl.when(sid == 0)
    def _():
        pltpu.sync_copy(out_sh, outp_hbm.at[cid])


def _sc_pass2(src, dst, xl, aexp, den):
    mesh = plsc.VectorSubcoreMesh(core_axis_name="c", subcore_axis_name="s",
                                  num_cores=NC, num_subcores=NS)
    return pl.kernel(
        _sc_pass2_body,
        out_type=jax.ShapeDtypeStruct((NC, NP, HID), jnp.float32),
        mesh=mesh,
        compiler_params=pltpu.CompilerParams(use_tc_tiling_on_sc=False, needs_layout_passes=False),
        scratch_types=[
            pltpu.VMEM((B,), jnp.int32),
            pltpu.VMEM((B,), jnp.int32),
            pltpu.VMEM((B, DM), jnp.float32),
            pltpu.VMEM((B, 16), jnp.float32),
            pltpu.VMEM((B, 16), jnp.float32),
            pltpu.VMEM((B, 16), jnp.float32),
            pltpu.VMEM((B, HID), jnp.float32),
            pltpu.VMEM((ZR2, HID), jnp.float32),
            pltpu.VMEM_SHARED((NP, HID), jnp.float32),
            pltpu.SemaphoreType.DMA,
            pltpu.SemaphoreType.DMA,
            pltpu.SemaphoreType.DMA,
        ],
    )(src, dst, xl, aexp, den)


def _dense_post_body(p0_ref, p1_ref, gb_ref, wc_ref, bc_ref, out_ref):
    o = (p0_ref[...] + p1_ref[...]) * (1.0 / HEADS) + gb_ref[...]
    o = jnp.maximum(o, 0.0)
    out_ref[...] = o @ wc_ref[...] + bc_ref[...]


def _dense_post(p0, p1, gb, wc_pad, bc_pad):
    full = lambda i: (0, 0)
    blk = lambda i: (i, 0)
    return pl.pallas_call(
        _dense_post_body,
        grid=(N_BLKS,),
        in_specs=[
            pl.BlockSpec((ROWS_BLK, HID), blk),
            pl.BlockSpec((ROWS_BLK, HID), blk),
            pl.BlockSpec((1, HID), full),
            pl.BlockSpec((HID, HID), full),
            pl.BlockSpec((1, HID), full),
        ],
        out_specs=pl.BlockSpec((ROWS_BLK, HID), blk),
        out_shape=jax.ShapeDtypeStruct((NP, HID), jnp.float32),
    )(p0, p1, gb, wc_pad, bc_pad)


def kernel(x, edge_index, W_in, b_in, prototypes, W_l, b_l, W_r, b_r, att,
           gat_bias, W_cls, b_cls):
    f32 = jnp.float32
    xp = jnp.zeros((NP, D_IN), f32).at[:N_NODES].set(x)

    loop = jnp.arange(N_NODES, dtype=jnp.int32)
    n_dummy = E_PAD - (edge_index.shape[1] + N_NODES)
    dummy = jnp.full((n_dummy,), N_NODES, jnp.int32)
    src = jnp.concatenate([edge_index[0].astype(jnp.int32), loop, dummy])
    dst = jnp.concatenate([edge_index[1].astype(jnp.int32), loop, dummy])

    wlm, wls = W_l[:HID], W_l[HID:]
    wrm, wrs = W_r[:HID], W_r[HID:]

    xl, xr, aself = _dense_pre(
        xp, W_in, b_in.reshape(1, HID), prototypes,
        wlm, wls, b_l.reshape(1, DM), wrm, wrs, b_r.reshape(1, DM), att)

    attb = jnp.broadcast_to(att.reshape(DM, 1), (DM, L)).reshape(-1)

    aexp, den = _sc_pass1(src, dst, xl, xr, aself, attb)
    outp = _sc_pass2(src, dst, xl, aexp, den)

    wc_pad = jnp.zeros((HID, HID), f32).at[:, :W_cls.shape[1]].set(W_cls)
    bc_pad = jnp.zeros((1, HID), f32).at[0, :b_cls.shape[0]].set(b_cls)
    res = _dense_post(outp[0], outp[1], gat_bias.reshape(1, HID),
                      wc_pad, bc_pad)
    return res[:N_NODES, :W_cls.shape[1]]


# speedup vs baseline: 13.1499x; 3.6337x over previous
import functools

import jax
import jax.numpy as jnp
from jax import lax
from jax.experimental import pallas as pl
from jax.experimental.pallas import tpu as pltpu
from jax.experimental.pallas import tpu_sc as plsc

N_NODES = 10000
D_IN = 128
HID = 128
HEADS = 4
DM = HEADS * HID
NEG = 0.2

NP = 10240
ROWS_BLK = 1280
N_BLKS = NP // ROWS_BLK

NC, NS, L = 2, 16, 16
NW = NC * NS
EP_TILE = 5376
E_PAD = NW * EP_TILE
B = 64
NB = EP_TILE // B
G = B // L
ZR = 256
ZR2 = 32


def _dense_pre_body(x_ref, w_in_ref, b_in_ref, pro_ref, wlm_ref, wls_ref,
                    bl_ref, wrm_ref, wrs_ref, br_ref, att_ref,
                    xl_ref, xr_ref, aself_ref):
    xb = x_ref[...]
    h = jnp.maximum(xb @ w_in_ref[...] + b_in_ref[...], 0.0)
    hn = h / (jnp.sqrt(jnp.sum(h * h, axis=1, keepdims=True)) + 1e-12)
    pro = pro_ref[...]
    pn = pro / (jnp.sqrt(jnp.sum(pro * pro, axis=1, keepdims=True)) + 1e-12)
    sem = hn @ pn.T
    xl = h @ wlm_ref[...] + sem @ wls_ref[...] + bl_ref[...]
    xr = h @ wrm_ref[...] + sem @ wrs_ref[...] + br_ref[...]
    xl_ref[...] = xl
    xr_ref[...] = xr
    s = xl + xr
    lr = jnp.maximum(s, NEG * s)
    att = att_ref[...]
    parts = [
        jnp.sum(lr[:, h_ * HID:(h_ + 1) * HID] * att[h_][None, :],
                axis=1, keepdims=True)
        for h_ in range(HEADS)
    ]
    parts.append(jnp.zeros((ROWS_BLK, 16 - HEADS), jnp.float32))
    aself_ref[...] = jnp.concatenate(parts, axis=1)


def _dense_pre(xp, w_in, b_in, pro, wlm, wls, bl, wrm, wrs, br, att):
    full = lambda i: (0, 0)
    blk = lambda i: (i, 0)
    return pl.pallas_call(
        _dense_pre_body,
        grid=(N_BLKS,),
        in_specs=[
            pl.BlockSpec((ROWS_BLK, D_IN), blk),
            pl.BlockSpec((D_IN, HID), full),
            pl.BlockSpec((1, HID), full),
            pl.BlockSpec((2, HID), full),
            pl.BlockSpec((HID, DM), full),
            pl.BlockSpec((2, DM), full),
            pl.BlockSpec((1, DM), full),
            pl.BlockSpec((HID, DM), full),
            pl.BlockSpec((2, DM), full),
            pl.BlockSpec((1, DM), full),
            pl.BlockSpec((HEADS, HID), full),
        ],
        out_specs=[
            pl.BlockSpec((ROWS_BLK, DM), blk),
            pl.BlockSpec((ROWS_BLK, DM), blk),
            pl.BlockSpec((ROWS_BLK, 16), blk),
        ],
        out_shape=[
            jax.ShapeDtypeStruct((NP, DM), jnp.float32),
            jax.ShapeDtypeStruct((NP, DM), jnp.float32),
            jax.ShapeDtypeStruct((NP, 16), jnp.float32),
        ],
    )(xp, w_in, b_in, pro, wlm, wls, bl, wrm, wrs, br, att)


def _sc_pass1_body(src_hbm, dst_hbm, xl_hbm, xr_hbm, aself_hbm, attb_hbm,
                   aexp_hbm, den_hbm,
                   src_v, dst_v, xl_rows, xr_rows, aself_rows, aexp_buf,
                   attb_v, zbuf, den_sh, sem1, sem2, sem3):
    cid = lax.axis_index("c")
    sid = lax.axis_index("s")
    wid = cid * NS + sid

    pltpu.sync_copy(attb_hbm, attb_v)

    zero16 = jnp.zeros((L,), jnp.float32)

    @pl.loop(0, B)
    def _(i):
        aexp_buf[i, :] = zero16

    @pl.loop(0, ZR)
    def _(i):
        zbuf[i, :] = zero16

    @pl.when(sid == 0)
    def _():
        @pl.loop(0, NP // ZR)
        def _(k):
            pltpu.sync_copy(zbuf, den_sh.at[pl.ds(k * ZR, ZR)])

    plsc.subcore_barrier()

    riota = lax.iota(jnp.int32, L)
    rows = [riota + g * L for g in range(G)]

    @pl.loop(0, NB)
    def _(b):
        base = wid * EP_TILE + b * B
        pltpu.sync_copy(src_hbm.at[pl.ds(base, B)], src_v)
        pltpu.sync_copy(dst_hbm.at[pl.ds(base, B)], dst_v)
        cp1 = pltpu.async_copy(xl_hbm.at[src_v], xl_rows, sem1)
        cp2 = pltpu.async_copy(xr_hbm.at[dst_v], xr_rows, sem2)
        cp3 = pltpu.async_copy(aself_hbm.at[dst_v], aself_rows, sem3)
        cp1.wait()
        cp2.wait()
        cp3.wait()

        for h in range(HEADS):
            init = tuple(jnp.zeros((L,), jnp.float32) for _ in range(G))

            @pl.loop(0, HID, init_carry=init, unroll=2)
            def accs(c, accs):
                attv = attb_v[pl.ds((h * HID + c) * L, L)]
                col = ((riota + c) & (HID - 1)) + h * HID
                out = []
                for g in range(G):
                    xlv = plsc.load_gather(xl_rows, [rows[g], col])
                    xrv = plsc.load_gather(xr_rows, [rows[g], col])
                    s = xlv + xrv
                    lr = jnp.maximum(s, NEG * s)
                    out.append(accs[g] + attv * lr)
                return tuple(out)

            hcol = jnp.full((L,), h, jnp.int32)
            for g in range(G):
                aself_v = plsc.load_gather(aself_rows, [rows[g], hcol])
                av = jnp.exp(accs[g] - aself_v)
                plsc.store_scatter(aexp_buf, [rows[g], hcol], av)

        pltpu.sync_copy(aexp_buf, aexp_hbm.at[pl.ds(base, B)])
        pltpu.sync_copy(aexp_buf, den_sh.at[dst_v], add=True)

    plsc.subcore_barrier()

    @pl.when(sid == 0)
    def _():
        pltpu.sync_copy(den_sh, den_hbm.at[cid])


def _sc_pass1(src, dst, xl, xr, aself, attb):
    mesh = plsc.VectorSubcoreMesh(core_axis_name="c", subcore_axis_name="s",
                                  num_cores=NC, num_subcores=NS)
    return pl.kernel(
        _sc_pass1_body,
        out_type=[
            jax.ShapeDtypeStruct((E_PAD, 16), jnp.float32),
            jax.ShapeDtypeStruct((NC, NP, 16), jnp.float32),
        ],
        mesh=mesh,
        compiler_params=pltpu.CompilerParams(use_tc_tiling_on_sc=False, needs_layout_passes=False),
        scratch_types=[
            pltpu.VMEM((B,), jnp.int32),
            pltpu.VMEM((B,), jnp.int32),
            pltpu.VMEM((B, DM), jnp.float32),
            pltpu.VMEM((B, DM), jnp.float32),
            pltpu.VMEM((B, 16), jnp.float32),
            pltpu.VMEM((B, 16), jnp.float32),
            pltpu.VMEM((HEADS * HID * L,), jnp.float32),
            pltpu.VMEM((ZR, 16), jnp.float32),
            pltpu.VMEM_SHARED((NP, 16), jnp.float32),
            pltpu.SemaphoreType.DMA,
            pltpu.SemaphoreType.DMA,
            pltpu.SemaphoreType.DMA,
        ],
    )(src, dst, xl, xr, aself, attb)


def _sc_pass2_body(src_hbm, dst_hbm, xl_hbm, aexp_hbm, den_hbm,
                   outp_hbm,
                   src_v, dst_v, xl_rows, aexp_rows, den0, den1, m_rows,
                   zbuf, out_sh, sem1, sem2, sem3):
    cid = lax.axis_index("c")
    sid = lax.axis_index("s")
    wid = cid * NS + sid

    zero128 = jnp.zeros((L,), jnp.float32)

    @pl.loop(0, ZR2)
    def _(i):
        for k in range(HID // L):
            zbuf[i, pl.ds(k * L, L)] = zero128

    @pl.when(sid == 0)
    def _():
        @pl.loop(0, NP // ZR2)
        def _(k):
            pltpu.sync_copy(zbuf, out_sh.at[pl.ds(k * ZR2, ZR2)])

    plsc.subcore_barrier()

    riota = lax.iota(jnp.int32, L)
    rows = [riota + g * L for g in range(G)]

    @pl.loop(0, NB)
    def _(b):
        base = wid * EP_TILE + b * B
        pltpu.sync_copy(src_hbm.at[pl.ds(base, B)], src_v)
        pltpu.sync_copy(dst_hbm.at[pl.ds(base, B)], dst_v)
        pltpu.sync_copy(aexp_hbm.at[pl.ds(base, B)], aexp_rows)
        cp1 = pltpu.async_copy(xl_hbm.at[src_v], xl_rows, sem1)
        cp2 = pltpu.async_copy(den_hbm.at[0].at[dst_v], den0, sem2)
        cp3 = pltpu.async_copy(den_hbm.at[1].at[dst_v], den1, sem3)
        cp1.wait()
        cp2.wait()
        cp3.wait()

        for g in range(G):
            wv = []
            for h in range(HEADS):
                hcol = jnp.full((L,), h, jnp.int32)
                av = plsc.load_gather(aexp_rows, [rows[g], hcol])
                d0 = plsc.load_gather(den0, [rows[g], hcol])
                d1 = plsc.load_gather(den1, [rows[g], hcol])
                wv.append(av / (d0 + d1 + 1e-16))

            @pl.loop(0, HID, unroll=2)
            def _(c):
                ch = (riota + c) & (HID - 1)
                acc = jnp.zeros((L,), jnp.float32)
                for h in range(HEADS):
                    xlv = plsc.load_gather(xl_rows, [rows[g], ch + h * HID])
                    acc = acc + wv[h] * xlv
                plsc.store_scatter(m_rows, [rows[g], ch], acc)

        pltpu.sync_copy(m_rows, out_sh.at[dst_v], add=True)

    plsc.subcore_barrier()

    @pl.when(sid == 0)
    def _():
        pltpu.sync_copy(out_sh, outp_hbm.at[cid])


def _sc_pass2(src, dst, xl, aexp, den):
    mesh = plsc.VectorSubcoreMesh(core_axis_name="c", subcore_axis_name="s",
                                  num_cores=NC, num_subcores=NS)
    return pl.kernel(
        _sc_pass2_body,
        out_type=jax.ShapeDtypeStruct((NC, NP, HID), jnp.float32),
        mesh=mesh,
        compiler_params=pltpu.CompilerParams(use_tc_tiling_on_sc=False, needs_layout_passes=False),
        scratch_types=[
            pltpu.VMEM((B,), jnp.int32),
            pltpu.VMEM((B,), jnp.int32),
            pltpu.VMEM((B, DM), jnp.float32),
            pltpu.VMEM((B, 16), jnp.float32),
            pltpu.VMEM((B, 16), jnp.float32),
            pltpu.VMEM((B, 16), jnp.float32),
            pltpu.VMEM((B, HID), jnp.float32),
            pltpu.VMEM((ZR2, HID), jnp.float32),
            pltpu.VMEM_SHARED((NP, HID), jnp.float32),
            pltpu.SemaphoreType.DMA,
            pltpu.SemaphoreType.DMA,
            pltpu.SemaphoreType.DMA,
        ],
    )(src, dst, xl, aexp, den)


def _dense_post_body(p0_ref, p1_ref, gb_ref, wc_ref, bc_ref, out_ref):
    o = (p0_ref[...] + p1_ref[...]) * (1.0 / HEADS) + gb_ref[...]
    o = jnp.maximum(o, 0.0)
    out_ref[...] = o @ wc_ref[...] + bc_ref[...]


def _dense_post(p0, p1, gb, wc_pad, bc_pad):
    full = lambda i: (0, 0)
    blk = lambda i: (i, 0)
    return pl.pallas_call(
        _dense_post_body,
        grid=(N_BLKS,),
        in_specs=[
            pl.BlockSpec((ROWS_BLK, HID), blk),
            pl.BlockSpec((ROWS_BLK, HID), blk),
            pl.BlockSpec((1, HID), full),
            pl.BlockSpec((HID, HID), full),
            pl.BlockSpec((1, HID), full),
        ],
        out_specs=pl.BlockSpec((ROWS_BLK, HID), blk),
        out_shape=jax.ShapeDtypeStruct((NP, HID), jnp.float32),
    )(p0, p1, gb, wc_pad, bc_pad)


def kernel(x, edge_index, W_in, b_in, prototypes, W_l, b_l, W_r, b_r, att,
           gat_bias, W_cls, b_cls):
    f32 = jnp.float32
    xp = jnp.zeros((NP, D_IN), f32).at[:N_NODES].set(x)

    loop = jnp.arange(N_NODES, dtype=jnp.int32)
    n_dummy = E_PAD - (edge_index.shape[1] + N_NODES)
    dummy = jnp.full((n_dummy,), N_NODES, jnp.int32)
    src = jnp.concatenate([edge_index[0].astype(jnp.int32), loop, dummy])
    dst = jnp.concatenate([edge_index[1].astype(jnp.int32), loop, dummy])

    wlm, wls = W_l[:HID], W_l[HID:]
    wrm, wrs = W_r[:HID], W_r[HID:]

    xl, xr, aself = _dense_pre(
        xp, W_in, b_in.reshape(1, HID), prototypes,
        wlm, wls, b_l.reshape(1, DM), wrm, wrs, b_r.reshape(1, DM), att)

    rot = (jnp.arange(HID)[:, None] + jnp.arange(L)[None, :]) % HID
    attb = att[:, rot].reshape(-1)

    aexp, den = _sc_pass1(src, dst, xl, xr, aself, attb)
    outp = _sc_pass2(src, dst, xl, aexp, den)

    wc_pad = jnp.zeros((HID, HID), f32).at[:, :W_cls.shape[1]].set(W_cls)
    bc_pad = jnp.zeros((1, HID), f32).at[0, :b_cls.shape[0]].set(b_cls)
    res = _dense_post(outp[0], outp[1], gat_bias.reshape(1, HID),
                      wc_pad, bc_pad)
    return res[:N_NODES, :W_cls.shape[1]]
